# baseline (speedup 1.0000x reference)
import functools

import jax
import jax.numpy as jnp
from jax import lax
from jax.experimental import pallas as pl
from jax.experimental.pallas import tpu as pltpu
from jax.experimental.pallas import tpu_sc as plsc

MARGIN = 0.1


@functools.lru_cache(maxsize=None)
def _sc_gather(T, B, N):
    info = plsc.get_sparse_core_info()
    NC, NS, L = info.num_cores, info.num_subcores, info.num_lanes
    NW = NC * NS
    chunk = (B * N) // NW
    assert (B * N) % NW == 0 and N % chunk == 0 and chunk % L == 0
    mesh = plsc.VectorSubcoreMesh(core_axis_name="c", subcore_axis_name="s")

    @functools.partial(
        pl.kernel,
        mesh=mesh,
        compiler_params=pltpu.CompilerParams(needs_layout_passes=False),
        out_type=jax.ShapeDtypeStruct((B * N,), jnp.float32),
        scratch_types=[
            pltpu.VMEM((T,), jnp.float32),
            pltpu.VMEM((B,), jnp.int32),
            pltpu.VMEM((chunk,), jnp.int32),
            pltpu.VMEM((chunk,), jnp.float32),
        ],
    )
    def sc_gather(scores_hbm, off_hbm, rank_hbm, out_hbm,
                  scores_v, off_v, idx_v, g_v):
        wid = lax.axis_index("s") * NC + lax.axis_index("c")
        base = wid * chunk
        pltpu.sync_copy(scores_hbm, scores_v)
        pltpu.sync_copy(off_hbm, off_v)
        pltpu.sync_copy(rank_hbm.at[pl.ds(base, chunk)], idx_v)
        b = base // N
        off_b = plsc.load_gather(off_v, [jnp.full((L,), b, jnp.int32)])
        for k in range(chunk // L):
            v = idx_v[pl.ds(k * L, L)] + off_b
            g_v[pl.ds(k * L, L)] = plsc.load_gather(scores_v, [v])
        pltpu.sync_copy(g_v, out_hbm.at[pl.ds(base, chunk)])

    return sc_gather


@functools.lru_cache(maxsize=None)
def _tc_loss(B, N, TI=256, interpret=False):
    NJ = N // TI
    assert N % TI == 0

    def body(grow_ref, gcol_ref, out_ref):
        b = pl.program_id(0)
        jt = pl.program_id(1)
        rowm = grow_ref[0] + MARGIN

        def offdiag(it, acc):
            col = gcol_ref[0, pl.ds(it * TI, TI), :]
            d = rowm - col
            return acc + jnp.sum(jnp.maximum(d, 0.0))

        acc = lax.fori_loop(0, jt, offdiag, 0.0)

        col = gcol_ref[0, pl.ds(jt * TI, TI), :]
        d = rowm - col
        li = lax.broadcasted_iota(jnp.int32, (TI, TI), 0)
        lj = lax.broadcasted_iota(jnp.int32, (TI, TI), 1)
        acc = acc + jnp.sum(jnp.where(lj > li, jnp.maximum(d, 0.0), 0.0))

        @pl.when((b == 0) & (jt == 0))
        def _init():
            out_ref[...] = jnp.zeros((1, 1), jnp.float32)

        out_ref[...] += jnp.reshape(acc, (1, 1))

    return pl.pallas_call(
        body,
        grid=(B, NJ),
        in_specs=[
            pl.BlockSpec((1, 1, TI), lambda b, jt: (b, 0, jt)),
            pl.BlockSpec((1, N, 1), lambda b, jt: (b, 0, 0)),
        ],
        out_specs=pl.BlockSpec((1, 1), lambda b, jt: (0, 0)),
        out_shape=jax.ShapeDtypeStruct((1, 1), jnp.float32),
        interpret=interpret,
    )


def kernel(scores, nBestIndex, werRank):
    B, N = werRank.shape
    T = scores.shape[0]
    nb = nBestIndex.astype(jnp.int32)
    off = jnp.cumsum(nb) - nb
    g = _sc_gather(T, B, N)(scores, off, werRank.reshape(B * N))
    g2 = g.reshape(B, N)
    loss = _tc_loss(B, N)(g2[:, None, :], g2[:, :, None])
    return loss[0, 0]


# speedup vs baseline: 1.7248x; 1.7248x over previous
import functools

import jax
import jax.numpy as jnp
from jax import lax
from jax.experimental import pallas as pl
from jax.experimental.pallas import tpu as pltpu
from jax.experimental.pallas import tpu_sc as plsc

MARGIN = 0.1


@functools.lru_cache(maxsize=None)
def _sc_gather(T, B, N):
    info = plsc.get_sparse_core_info()
    NC, NS, L = info.num_cores, info.num_subcores, info.num_lanes
    NW = NC * NS
    chunk = (B * N) // NW
    assert (B * N) % NW == 0 and N % chunk == 0 and chunk % L == 0
    mesh = plsc.VectorSubcoreMesh(core_axis_name="c", subcore_axis_name="s")

    @functools.partial(
        pl.kernel,
        mesh=mesh,
        compiler_params=pltpu.CompilerParams(needs_layout_passes=False),
        out_type=jax.ShapeDtypeStruct((B * N,), jnp.float32),
        scratch_types=[
            pltpu.VMEM((T,), jnp.float32),
            pltpu.VMEM((B,), jnp.int32),
            pltpu.VMEM((chunk,), jnp.int32),
            pltpu.VMEM((chunk,), jnp.float32),
        ],
    )
    def sc_gather(scores_hbm, off_hbm, rank_hbm, out_hbm,
                  scores_v, off_v, idx_v, g_v):
        wid = lax.axis_index("s") * NC + lax.axis_index("c")
        base = wid * chunk
        pltpu.sync_copy(scores_hbm, scores_v)
        pltpu.sync_copy(off_hbm, off_v)
        pltpu.sync_copy(rank_hbm.at[pl.ds(base, chunk)], idx_v)
        b = base // N
        off_b = plsc.load_gather(off_v, [jnp.full((L,), b, jnp.int32)])
        for k in range(chunk // L):
            v = idx_v[pl.ds(k * L, L)] + off_b
            g_v[pl.ds(k * L, L)] = plsc.load_gather(scores_v, [v])
        pltpu.sync_copy(g_v, out_hbm.at[pl.ds(base, chunk)])

    return sc_gather


@functools.lru_cache(maxsize=None)
def _tc_loss(B, N, TI=128, interpret=False):
    NJ = N // TI
    assert N % TI == 0 and TI % 8 == 0

    def body(grow_ref, gcol_ref, out_ref):
        b = pl.program_id(0)
        acc = jnp.zeros((8, TI), jnp.float32)
        for jt in range(NJ):
            rowm = grow_ref[0, :, pl.ds(jt * TI, TI)] + MARGIN
            for it in range(jt + 1):
                col = gcol_ref[0, pl.ds(it * TI, TI), :]
                d = jnp.maximum(rowm - col, 0.0)
                if it == jt:
                    li = lax.broadcasted_iota(jnp.int32, (TI, TI), 0)
                    lj = lax.broadcasted_iota(jnp.int32, (TI, TI), 1)
                    d = jnp.where(lj > li, d, 0.0)
                acc = acc + jnp.sum(d.reshape(TI // 8, 8, TI), axis=0)
        total = jnp.sum(acc)

        @pl.when(b == 0)
        def _init():
            out_ref[...] = jnp.zeros((1, 1), jnp.float32)

        out_ref[...] += jnp.reshape(total, (1, 1))

    return pl.pallas_call(
        body,
        grid=(B,),
        in_specs=[
            pl.BlockSpec((1, 1, N), lambda b: (b, 0, 0)),
            pl.BlockSpec((1, N, 1), lambda b: (b, 0, 0)),
        ],
        out_specs=pl.BlockSpec((1, 1), lambda b: (0, 0)),
        out_shape=jax.ShapeDtypeStruct((1, 1), jnp.float32),
        interpret=interpret,
    )


def kernel(scores, nBestIndex, werRank):
    B, N = werRank.shape
    T = scores.shape[0]
    nb = nBestIndex.astype(jnp.int32)
    off = jnp.cumsum(nb) - nb
    g = _sc_gather(T, B, N)(scores, off, werRank.reshape(B * N))
    g2 = g.reshape(B, N)
    loss = _tc_loss(B, N)(g2[:, None, :], g2[:, :, None])
    return loss[0, 0]


# speedup vs baseline: 2.4711x; 1.4327x over previous
import functools

import jax
import jax.numpy as jnp
from jax import lax
from jax.experimental import pallas as pl
from jax.experimental.pallas import tpu as pltpu
from jax.experimental.pallas import tpu_sc as plsc

MARGIN = 0.1


@functools.lru_cache(maxsize=None)
def _sc_gather(T, B, N):
    info = plsc.get_sparse_core_info()
    NC, NS, L = info.num_cores, info.num_subcores, info.num_lanes
    NW = NC * NS
    SR = N // 128
    assert NW >= B and N % 128 == 0 and 128 % L == 0
    mesh = plsc.VectorSubcoreMesh(core_axis_name="c", subcore_axis_name="s")

    @functools.partial(
        pl.kernel,
        mesh=mesh,
        compiler_params=pltpu.CompilerParams(needs_layout_passes=False),
        out_type=jax.ShapeDtypeStruct((B, SR, 128), jnp.float32),
        scratch_types=[
            pltpu.VMEM((T,), jnp.float32),
            pltpu.VMEM((B,), jnp.int32),
            pltpu.VMEM((N,), jnp.int32),
            pltpu.VMEM((SR, 128), jnp.float32),
        ],
    )
    def sc_gather(scores_hbm, off_hbm, rank_hbm, out_hbm,
                  scores_v, off_v, idx_v, g_v):
        wid = lax.axis_index("s") * NC + lax.axis_index("c")

        @pl.when(wid < B)
        def _():
            b = wid
            pltpu.sync_copy(scores_hbm, scores_v)
            pltpu.sync_copy(off_hbm, off_v)
            pltpu.sync_copy(rank_hbm.at[pl.ds(b * N, N)], idx_v)
            off_b = plsc.load_gather(off_v, [jnp.full((L,), b, jnp.int32)])
            for k in range(N // L):
                v = idx_v[pl.ds(k * L, L)] + off_b
                g_v[(k * L) // 128, pl.ds((k * L) % 128, L)] = (
                    plsc.load_gather(scores_v, [v])
                )
            pltpu.sync_copy(g_v, out_hbm.at[b])

    return sc_gather


@functools.lru_cache(maxsize=None)
def _tc_loss(B, N, interpret=False):
    SR = N // 128
    assert N % 128 == 0

    def body(g_ref, out_ref):
        b = pl.program_id(0)
        gmat = g_ref[0]
        gt = jnp.transpose(gmat)
        acc = jnp.zeros((8, 128), jnp.float32)
        for jt in range(SR):
            rowm = gmat[jt:jt + 1, :] + MARGIN
            for it in range(jt + 1):
                col = gt[:, it:it + 1]
                d = jnp.maximum(rowm - col, 0.0)
                if it == jt:
                    li = lax.broadcasted_iota(jnp.int32, (128, 128), 0)
                    lj = lax.broadcasted_iota(jnp.int32, (128, 128), 1)
                    d = jnp.where(lj > li, d, 0.0)
                acc = acc + jnp.sum(d.reshape(16, 8, 128), axis=0)
        total = jnp.sum(acc)

        @pl.when(b == 0)
        def _init():
            out_ref[...] = jnp.zeros((1, 1), jnp.float32)

        out_ref[...] += jnp.reshape(total, (1, 1))

    return pl.pallas_call(
        body,
        grid=(B,),
        in_specs=[pl.BlockSpec((1, SR, 128), lambda b: (b, 0, 0))],
        out_specs=pl.BlockSpec((1, 1), lambda b: (0, 0)),
        out_shape=jax.ShapeDtypeStruct((1, 1), jnp.float32),
        interpret=interpret,
    )


def kernel(scores, nBestIndex, werRank):
    B, N = werRank.shape
    T = scores.shape[0]
    nb = nBestIndex.astype(jnp.int32)
    off = jnp.cumsum(nb) - nb
    g = _sc_gather(T, B, N)(scores, off, werRank.reshape(B * N))
    loss = _tc_loss(B, N)(g)
    return loss[0, 0]
